# Initial kernel scaffold; baseline (speedup 1.0000x reference)
import jax
import jax.numpy as jnp
from jax.experimental import pallas as pl


def kernel(obs, hidden_states, matrix, obs_cW, obs_cb, obs_f1W, obs_f1b, obs_f2W, obs_f2b, hid_cW, hid_cb, hid_f1W, hid_f1b, hid_f2W, hid_f2b, gru_Wih, gru_Whh, gru_bih, gru_bhh, enc_W, enc_b):
    raise NotImplementedError("write your pallas kernel here")


# speedup vs baseline: 3577.6350x; 3577.6350x over previous
import jax
import jax.numpy as jnp
from jax.experimental import pallas as pl

N = 1024
OBS = 128
HID = 256
H = 256


def _encoder_body(obs_ref, hid_ref, mat_ref,
                  obs_cW_ref, obs_cb_ref, obs_f1W_ref, obs_f1b_ref,
                  obs_f2W_ref, obs_f2b_ref,
                  hid_cW_ref, hid_cb_ref, hid_f1W_ref, hid_f1b_ref,
                  hid_f2W_ref, hid_f2b_ref,
                  gru_Wih_ref, gru_Whh_ref, gru_bih_ref, gru_bhh_ref,
                  enc_W_ref, enc_b_ref,
                  latent_ref, next_hid_ref):
    mf = mat_ref[...].astype(jnp.float32)
    deg = jnp.sum(mf, axis=0) + 1.0
    dinv = jax.lax.rsqrt(deg)[:, None]

    def gcn_mlp(x, cW, cb, f1W, f1b, f2W, f2b):
        xw = jnp.dot(x, cW, preferred_element_type=jnp.float32)
        s = dinv * xw
        agg = jax.lax.dot_general(
            mf, s, (((0,), (0,)), ((), ())),
            preferred_element_type=jnp.float32) + s
        h = jnp.maximum(dinv * agg + cb, 0.0)
        h = jnp.maximum(jnp.dot(h, f1W, preferred_element_type=jnp.float32)
                        + f1b, 0.0)
        return jnp.dot(h, f2W, preferred_element_type=jnp.float32) + f2b

    phi = gcn_mlp(obs_ref[...], obs_cW_ref[...], obs_cb_ref[...],
                  obs_f1W_ref[...], obs_f1b_ref[...],
                  obs_f2W_ref[...], obs_f2b_ref[...])
    psi = gcn_mlp(hid_ref[...], hid_cW_ref[...], hid_cb_ref[...],
                  hid_f1W_ref[...], hid_f1b_ref[...],
                  hid_f2W_ref[...], hid_f2b_ref[...])

    gi = jax.lax.dot_general(
        phi, gru_Wih_ref[...], (((1,), (1,)), ((), ())),
        preferred_element_type=jnp.float32) + gru_bih_ref[...]
    gh = jax.lax.dot_general(
        psi, gru_Whh_ref[...], (((1,), (1,)), ((), ())),
        preferred_element_type=jnp.float32) + gru_bhh_ref[...]
    r = jax.nn.sigmoid(gi[:, :HID] + gh[:, :HID])
    z = jax.nn.sigmoid(gi[:, HID:2 * HID] + gh[:, HID:2 * HID])
    n = jnp.tanh(gi[:, 2 * HID:] + r * gh[:, 2 * HID:])
    next_hid = (1.0 - z) * n + z * psi

    latent_ref[...] = jnp.dot(next_hid, enc_W_ref[...],
                              preferred_element_type=jnp.float32) + enc_b_ref[...]
    next_hid_ref[...] = next_hid


def kernel(obs, hidden_states, matrix,
           obs_cW, obs_cb, obs_f1W, obs_f1b, obs_f2W, obs_f2b,
           hid_cW, hid_cb, hid_f1W, hid_f1b, hid_f2W, hid_f2b,
           gru_Wih, gru_Whh, gru_bih, gru_bhh,
           enc_W, enc_b):
    latent, next_hid = pl.pallas_call(
        _encoder_body,
        out_shape=(
            jax.ShapeDtypeStruct((N, H), jnp.float32),
            jax.ShapeDtypeStruct((N, HID), jnp.float32),
        ),
    )(obs, hidden_states, matrix,
      obs_cW, obs_cb, obs_f1W, obs_f1b, obs_f2W, obs_f2b,
      hid_cW, hid_cb, hid_f1W, hid_f1b, hid_f2W, hid_f2b,
      gru_Wih, gru_Whh, gru_bih, gru_bhh,
      enc_W, enc_b)
    return (latent, next_hid)
